# Initial kernel scaffold; baseline (speedup 1.0000x reference)
import jax
import jax.numpy as jnp
from jax.experimental import pallas as pl


def kernel(x):
    raise NotImplementedError("write your pallas kernel here")


# speedup vs baseline: 1.6575x; 1.6575x over previous
import functools

import jax
import jax.numpy as jnp
from jax import lax
from jax.experimental import pallas as pl
from jax.experimental.pallas import tpu as pltpu
from jax.experimental.pallas import tpu_sc as plsc

_B, _N = 64, 8192
_TOTAL = _B * _N
_NC, _NS, _L = 2, 16, 16
_NW = _NC * _NS
_PER_W = _TOTAL // _NW
_VECS = _PER_W // _L


def _body(x_hbm, out_hbm, xin_v, xout_v):
    wid = lax.axis_index("s") * _NC + lax.axis_index("c")
    base = wid * _PER_W
    pltpu.sync_copy(x_hbm.at[pl.ds(base, _PER_W)], xin_v)

    iot = lax.iota(jnp.int32, _L)
    off = iot & 3
    blk = iot - off

    dnums = lax.GatherDimensionNumbers(
        offset_dims=(), collapsed_slice_dims=(0,), start_index_map=(0,))

    def step(i, _):
        b = i * _L
        v = xin_v[pl.ds(b, _L)]
        beats = []
        for r in (1, 2, 3):
            noff = (off + r) & 3
            perm = blk | noff
            n = lax.gather(v, perm[:, None], dnums, (1,),
                           mode=lax.GatherScatterMode.PROMISE_IN_BOUNDS)
            lose = noff < off
            beats.append((n > v) | ((n == v) & lose))
        b1, b2, b3 = beats
        drop = (b1 & b2) | (b1 & b3) | (b2 & b3)
        xout_v[pl.ds(b, _L)] = jnp.where(drop, jnp.float32(0), v)
        return 0

    lax.fori_loop(0, _VECS, step, 0)
    pltpu.sync_copy(xout_v, out_hbm.at[pl.ds(base, _PER_W)])


@jax.jit
def kernel(x):
    mesh = plsc.VectorSubcoreMesh(core_axis_name="c", subcore_axis_name="s")
    fn = functools.partial(
        pl.kernel,
        mesh=mesh,
        out_type=jax.ShapeDtypeStruct((_TOTAL,), jnp.float32),
        scratch_types=[
            pltpu.VMEM((_PER_W,), jnp.float32),
            pltpu.VMEM((_PER_W,), jnp.float32),
        ],
    )(_body)
    return fn(x.reshape(_TOTAL)).reshape(_B, _N)


# speedup vs baseline: 1.7908x; 1.0804x over previous
import functools

import jax
import jax.numpy as jnp
from jax import lax
from jax.experimental import pallas as pl
from jax.experimental.pallas import tpu as pltpu
from jax.experimental.pallas import tpu_sc as plsc

_B, _N = 64, 8192
_NC, _NS, _L = 2, 16, 16
_RG, _CG = 8, 4
_RPW = _B // _RG
_CPW = _N // _CG
_VPR = _CPW // _L


def _body(x_hbm, out_hbm, xin_v, xout_v):
    wid = lax.axis_index("s") * _NC + lax.axis_index("c")
    rg = wid // _CG
    cg = wid - rg * _CG
    r0 = rg * _RPW
    c0 = cg * _CPW
    pltpu.sync_copy(x_hbm.at[pl.ds(r0, _RPW), pl.ds(c0, _CPW)], xin_v)

    iot = lax.iota(jnp.int32, _L)
    off = iot & 3
    blk = iot - off
    dnums = lax.GatherDimensionNumbers(
        offset_dims=(), collapsed_slice_dims=(0,), start_index_map=(0,))
    perms, loses = [], []
    for r in (1, 2, 3):
        noff = (off + r) & 3
        perms.append((blk | noff)[:, None])
        loses.append(noff < off)

    def step(i, _):
        c = i * _L
        for row in range(_RPW):
            v = xin_v[row, pl.ds(c, _L)]
            beats = []
            for perm, lose in zip(perms, loses):
                n = lax.gather(v, perm, dnums, (1,),
                               mode=lax.GatherScatterMode.PROMISE_IN_BOUNDS)
                beats.append((n > v) | ((n == v) & lose))
            b1, b2, b3 = beats
            drop = (b1 & b2) | (b3 & (b1 | b2))
            xout_v[row, pl.ds(c, _L)] = jnp.where(drop, jnp.float32(0), v)
        return 0

    lax.fori_loop(0, _VPR, step, 0)
    pltpu.sync_copy(xout_v, out_hbm.at[pl.ds(r0, _RPW), pl.ds(c0, _CPW)])


@jax.jit
def kernel(x):
    mesh = plsc.VectorSubcoreMesh(core_axis_name="c", subcore_axis_name="s")
    fn = functools.partial(
        pl.kernel,
        mesh=mesh,
        out_type=jax.ShapeDtypeStruct((_B, _N), jnp.float32),
        scratch_types=[
            pltpu.VMEM((_RPW, _CPW), jnp.float32),
            pltpu.VMEM((_RPW, _CPW), jnp.float32),
        ],
        compiler_params=pltpu.CompilerParams(use_tc_tiling_on_sc=True),
    )(_body)
    return fn(x)
